# Initial kernel scaffold; baseline (speedup 1.0000x reference)
import jax
import jax.numpy as jnp
from jax.experimental import pallas as pl


def kernel(x, y, embedding_table):
    raise NotImplementedError("write your pallas kernel here")


# speedup vs baseline: 2.0392x; 2.0392x over previous
import jax
import jax.numpy as jnp
from jax import lax
from jax.experimental import pallas as pl
from jax.experimental.pallas import tpu as pltpu
from jax.experimental.pallas import tpu_sc as plsc

_B, _T, _V = 32, 9, 8192
_N = _B * _T
_NC, _NS = 2, 16
_NW = _NC * _NS
_RPW = _N // _NW


_RPC = 8
_NCHUNK = _N // _RPC


def _sc_gather_body(table_hbm, xf_hbm, out_hbm, my_idx, rows, sem):
    wid = lax.axis_index("s") * _NC + lax.axis_index("c")

    def do_chunk(base):
        pltpu.sync_copy(xf_hbm.at[pl.ds(base, _RPC)], my_idx)
        pltpu.async_copy(table_hbm.at[my_idx], rows, sem).wait()
        pltpu.sync_copy(rows, out_hbm.at[pl.ds(base, _RPC)])

    do_chunk(wid * _RPC)

    @pl.when(wid < _NCHUNK - _NW)
    def _():
        do_chunk(_NW * _RPC + wid * _RPC)


def _sc_gather(table, xf):
    mesh = plsc.VectorSubcoreMesh(core_axis_name="c", subcore_axis_name="s")
    f = pl.kernel(
        _sc_gather_body,
        out_type=jax.ShapeDtypeStruct((_N, _V), jnp.float32),
        mesh=mesh,
        scratch_types=[
            pltpu.VMEM((_RPC,), jnp.int32),
            pltpu.VMEM((_RPC, _V), jnp.float32),
            pltpu.SemaphoreType.DMA,
        ],
    )
    return f(table, xf)


_ROWS_PER_BLK = 32
_NBLK = _N // _ROWS_PER_BLK


def _tc_loss_body(lg_ref, y_ref, loss_ref, acc_ref):
    i = pl.program_id(0)
    lg = lg_ref[...]
    m = jnp.max(lg, axis=1)
    s = jnp.sum(jnp.exp(lg - m[:, None]), axis=1)
    ids = lax.broadcasted_iota(jnp.int32, (_ROWS_PER_BLK, _V), 1)
    t = jnp.sum(jnp.where(ids == y_ref[...], lg, 0.0), axis=1)
    part = jnp.sum(t - m - jnp.log(s))

    @pl.when(i == 0)
    def _():
        acc_ref[0] = 0.0

    acc_ref[0] += part

    @pl.when(i == _NBLK - 1)
    def _():
        loss_ref[0, 0] = -acc_ref[0] / _N


def _tc_loss(logits, y2):
    return pl.pallas_call(
        _tc_loss_body,
        grid=(_NBLK,),
        in_specs=[
            pl.BlockSpec((_ROWS_PER_BLK, _V), lambda i: (i, 0)),
            pl.BlockSpec((_ROWS_PER_BLK, 1), lambda i: (i, 0)),
        ],
        out_specs=pl.BlockSpec(memory_space=pltpu.SMEM),
        out_shape=jax.ShapeDtypeStruct((1, 1), jnp.float32),
        scratch_shapes=[pltpu.SMEM((1,), jnp.float32)],
    )(logits, y2)


def kernel(x, y, embedding_table):
    xf = x.reshape(_N).astype(jnp.int32)
    logits = _sc_gather(embedding_table, xf)
    y2 = y.reshape(_N, 1).astype(jnp.int32)
    loss = _tc_loss(logits, y2)
    return (logits, loss.reshape(()))
